# Initial kernel scaffold; baseline (speedup 1.0000x reference)
import jax
import jax.numpy as jnp
from jax.experimental import pallas as pl


def kernel(features, labels, centers):
    raise NotImplementedError("write your pallas kernel here")


# speedup vs baseline: 3.5167x; 3.5167x over previous
import functools

import jax
import jax.numpy as jnp
from jax.experimental import pallas as pl
from jax.experimental.pallas import tpu as pltpu

BATCH = 16384
FEAT = 640
NCLASS = 6
CPAD = 8
BB = 2048
NB = BATCH // BB


def _tc_body(f_ref, lab_ref, ct_ref, out_ref, acc_ref):
    i = pl.program_id(0)

    @pl.when(i == 0)
    def _():
        acc_ref[0] = 0.0
        acc_ref[1] = 0.0

    f = f_ref[...]
    lab = lab_ref[...]
    ct = ct_ref[...]

    mask = (lab < NCLASS).astype(jnp.float32)
    onehot = (lab == jax.lax.broadcasted_iota(jnp.int32, (BB, CPAD), 1))
    onehot = onehot.astype(jnp.float32) * mask

    p = jnp.dot(f, ct, preferred_element_type=jnp.float32)
    c2 = jnp.sum(ct * ct, axis=0, keepdims=True)
    rows2 = jnp.sum(f * f, axis=1, keepdims=True)

    contrib = jnp.sum(rows2 * mask) + jnp.sum(onehot * (c2 - 2.0 * p))
    acc_ref[0] += contrib
    acc_ref[1] += jnp.sum(mask)

    @pl.when(i == NB - 1)
    def _():
        out_ref[0, 0] = acc_ref[0] / acc_ref[1]


@jax.jit
def _center_loss_tc(features, labels, centers_t):
    lab2d = labels.reshape(BATCH, 1)
    out = pl.pallas_call(
        _tc_body,
        grid=(NB,),
        in_specs=[
            pl.BlockSpec((BB, FEAT), lambda i: (i, 0)),
            pl.BlockSpec((BB, 1), lambda i: (i, 0)),
            pl.BlockSpec((FEAT, CPAD), lambda i: (0, 0)),
        ],
        out_specs=pl.BlockSpec(memory_space=pltpu.SMEM),
        out_shape=jax.ShapeDtypeStruct((1, 1), jnp.float32),
        scratch_shapes=[pltpu.SMEM((2,), jnp.float32)],
    )(features, lab2d, centers_t)
    return out[0, 0]


def kernel(features, labels, centers):
    centers_t = jnp.zeros((FEAT, CPAD), jnp.float32).at[:, :NCLASS].set(centers.T)
    return _center_loss_tc(features, labels, centers_t)
